# Initial kernel scaffold; baseline (speedup 1.0000x reference)
import functools

import jax
import jax.numpy as jnp
from jax import lax
from jax.experimental import pallas as pl
from jax.experimental.pallas import tpu as pltpu
from jax.experimental.pallas import tpu_sc as plsc

N = 1048576
M = 16384
D = 64

_INFO = plsc.get_sparse_core_info()
_NC = _INFO.num_cores
_NS = _INFO.num_subcores
_NW = _NC * _NS
_CHUNK = N // _NW
_LANES = 16


def _sc_gather(total_charge, idx):
    mesh = plsc.VectorSubcoreMesh(core_axis_name="c", subcore_axis_name="s")

    @functools.partial(
        pl.kernel,
        mesh=mesh,
        out_type=jax.ShapeDtypeStruct((N,), jnp.float32),
        scratch_types=[
            pltpu.VMEM((M,), jnp.float32),
            pltpu.VMEM((_CHUNK,), jnp.int32),
            pltpu.VMEM((_CHUNK,), jnp.float32),
        ],
    )
    def k(tc_hbm, idx_hbm, out_hbm, table_v, idx_v, val_v):
        wid = lax.axis_index("s") * _NC + lax.axis_index("c")
        base = wid * _CHUNK
        pltpu.sync_copy(tc_hbm, table_v)
        pltpu.sync_copy(idx_hbm.at[pl.ds(base, _CHUNK)], idx_v)

        def body(i, carry):
            off = i * _LANES
            iv = idx_v[pl.ds(off, _LANES)]
            val_v[pl.ds(off, _LANES)] = plsc.load_gather(table_v, [iv])
            return carry

        lax.fori_loop(0, _CHUNK // _LANES, body, 0, unroll=8)
        pltpu.sync_copy(val_v, out_hbm.at[pl.ds(base, _CHUNK)])

    return k(total_charge, idx)


def _tc_concat(x, g2d):
    bn = 4096
    grid = (N // bn,)

    def body(x_ref, g_ref, o_ref):
        o_ref[:, :D] = x_ref[...]
        o_ref[:, D:] = g_ref[...]

    return pl.pallas_call(
        body,
        grid=grid,
        in_specs=[
            pl.BlockSpec((bn, D), lambda i: (i, 0)),
            pl.BlockSpec((bn, 1), lambda i: (i, 0)),
        ],
        out_specs=pl.BlockSpec((bn, D + 1), lambda i: (i, 0)),
        out_shape=jax.ShapeDtypeStruct((N, D + 1), jnp.float32),
        compiler_params=pltpu.CompilerParams(
            dimension_semantics=("arbitrary",),
        ),
    )(x, g2d)


def kernel(per_atom_property_tensor, total_charge, atomic_subsystem_indices):
    idx = atomic_subsystem_indices.astype(jnp.int32)
    g = _sc_gather(total_charge, idx)
    return _tc_concat(per_atom_property_tensor, g.reshape(N, 1))


# speedup vs baseline: 7.1565x; 7.1565x over previous
import functools

import jax
import jax.numpy as jnp
from jax import lax
from jax.experimental import pallas as pl
from jax.experimental.pallas import tpu as pltpu
from jax.experimental.pallas import tpu_sc as plsc

N = 1048576
M = 16384
D = 64

_NC = 2
_NS = 16
_NW = _NC * _NS
_CHUNK = N // _NW
_LANES = 16


def _sc_gather(total_charge, idx):
    mesh = plsc.VectorSubcoreMesh(core_axis_name="c", subcore_axis_name="s")

    @functools.partial(
        pl.kernel,
        mesh=mesh,
        out_type=jax.ShapeDtypeStruct((N,), jnp.float32),
        scratch_types=[
            pltpu.VMEM((M,), jnp.float32),
            pltpu.VMEM((_CHUNK,), jnp.int32),
            pltpu.VMEM((_CHUNK,), jnp.float32),
        ],
        compiler_params=pltpu.CompilerParams(needs_layout_passes=False),
    )
    def k(tc_hbm, idx_hbm, out_hbm, table_v, idx_v, val_v):
        wid = lax.axis_index("s") * _NC + lax.axis_index("c")
        base = wid * _CHUNK
        pltpu.sync_copy(tc_hbm, table_v)
        pltpu.sync_copy(idx_hbm.at[pl.ds(base, _CHUNK)], idx_v)

        def body(i, carry):
            off = i * _LANES
            iv = idx_v[pl.ds(off, _LANES)]
            val_v[pl.ds(off, _LANES)] = plsc.load_gather(table_v, [iv])
            return carry

        lax.fori_loop(0, _CHUNK // _LANES, body, 0, unroll=8)
        pltpu.sync_copy(val_v, out_hbm.at[pl.ds(base, _CHUNK)])

    return k(total_charge, idx)


def _tc_concat(x, g2d):
    bn = 4096
    grid = (N // bn,)

    def body(x_ref, g_ref, o_ref):
        o_ref[:, :D] = x_ref[...]
        o_ref[:, D:] = g_ref[...]

    return pl.pallas_call(
        body,
        grid=grid,
        in_specs=[
            pl.BlockSpec((bn, D), lambda i: (i, 0)),
            pl.BlockSpec((bn, 1), lambda i: (i, 0)),
        ],
        out_specs=pl.BlockSpec((bn, D + 1), lambda i: (i, 0)),
        out_shape=jax.ShapeDtypeStruct((N, D + 1), jnp.float32),
        compiler_params=pltpu.CompilerParams(
            dimension_semantics=("arbitrary",),
        ),
    )(x, g2d)


def kernel(per_atom_property_tensor, total_charge, atomic_subsystem_indices):
    idx = atomic_subsystem_indices.astype(jnp.int32)
    g = _sc_gather(total_charge, idx)
    return _tc_concat(per_atom_property_tensor, g.reshape(N, 1))


# speedup vs baseline: 9.1424x; 1.2775x over previous
import functools

import jax
import jax.numpy as jnp
from jax import lax
from jax.experimental import pallas as pl
from jax.experimental.pallas import tpu as pltpu
from jax.experimental.pallas import tpu_sc as plsc

N = 1048576
M = 16384
D = 64

_NC = 2
_NS = 16
_NW = _NC * _NS
_CHUNK = N // _NW
_LANES = 16


def _sc_gather(total_charge, idx):
    mesh = plsc.VectorSubcoreMesh(core_axis_name="c", subcore_axis_name="s")

    @functools.partial(
        pl.kernel,
        mesh=mesh,
        out_type=jax.ShapeDtypeStruct((N,), jnp.float32),
        scratch_types=[
            pltpu.VMEM((M,), jnp.float32),
            pltpu.VMEM((_CHUNK,), jnp.int32),
            pltpu.VMEM((_CHUNK,), jnp.float32),
        ],
        compiler_params=pltpu.CompilerParams(needs_layout_passes=False),
    )
    def k(tc_hbm, idx_hbm, out_hbm, table_v, idx_v, val_v):
        wid = lax.axis_index("s") * _NC + lax.axis_index("c")
        base = wid * _CHUNK
        pltpu.sync_copy(tc_hbm, table_v)
        pltpu.sync_copy(idx_hbm.at[pl.ds(base, _CHUNK)], idx_v)

        def body(i, carry):
            off = i * _LANES
            iv = idx_v[pl.ds(off, _LANES)]
            val_v[pl.ds(off, _LANES)] = plsc.load_gather(table_v, [iv])
            return carry

        lax.fori_loop(0, _CHUNK // _LANES, body, 0, unroll=8)
        pltpu.sync_copy(val_v, out_hbm.at[pl.ds(base, _CHUNK)])

    return k(total_charge, idx)


def _tc_concat(x, g_rows):
    bn = 4096
    gr = bn // 128
    grid = (N // bn,)

    def body(x_ref, g_ref, o_ref):
        o_ref[:, :D] = x_ref[...]
        t = jnp.transpose(g_ref[...])
        for u in range(gr):
            o_ref[pl.ds(128 * u, 128), D:] = t[:, u : u + 1]

    return pl.pallas_call(
        body,
        grid=grid,
        in_specs=[
            pl.BlockSpec((bn, D), lambda i: (i, 0)),
            pl.BlockSpec((gr, 128), lambda i: (i, 0)),
        ],
        out_specs=pl.BlockSpec((bn, D + 1), lambda i: (i, 0)),
        out_shape=jax.ShapeDtypeStruct((N, D + 1), jnp.float32),
        compiler_params=pltpu.CompilerParams(
            dimension_semantics=("arbitrary",),
        ),
    )(x, g_rows)


def kernel(per_atom_property_tensor, total_charge, atomic_subsystem_indices):
    idx = atomic_subsystem_indices.astype(jnp.int32)
    g = _sc_gather(total_charge, idx)
    return _tc_concat(per_atom_property_tensor, g.reshape(N // 128, 128))
